# baseline (speedup 1.0000x reference)
import functools

import jax
import jax.numpy as jnp
from jax import lax
from jax.experimental import pallas as pl
from jax.experimental.pallas import tpu as pltpu
from jax.experimental.pallas import tpu_sc as plsc

N = 10000
E = 320000
D_IN = 128
DH = 16
NC = 2
NS = 16
NW = NC * NS
CHUNK = 128
NBUF = 4
C = 80
CA = C + NBUF
EPW = C * CHUNK
E_PAD = NW * EPW
N_ACC = 10112
RPT = N_ACC // NS

_mesh = plsc.VectorSubcoreMesh(core_axis_name="c", subcore_axis_name="s")
_acc_ty = jax.ShapeDtypeStruct((NC, N_ACC, DH), jnp.float32)
_sc_params = pltpu.CompilerParams(use_tc_tiling_on_sc=False)


@functools.partial(
    pl.kernel,
    out_type=_acc_ty,
    mesh=_mesh,
    scratch_types=[
        pltpu.VMEM((C, CHUNK), jnp.int32),
        pltpu.VMEM((CHUNK, DH), jnp.float32),
        pltpu.VMEM_SHARED((N_ACC, DH), jnp.float32),
    ],
    compiler_params=_sc_params,
)
def _sc_degree(dst_hbm, ones_hbm, zeros_hbm, out_hbm, dst_v, ones_v, acc):
    cid = lax.axis_index("c")
    sid = lax.axis_index("s")
    wid = sid * NC + cid
    r0 = sid * RPT
    pltpu.sync_copy(zeros_hbm.at[pl.ds(r0, RPT)], acc.at[pl.ds(r0, RPT)])
    pltpu.sync_copy(dst_hbm.at[wid], dst_v)
    pltpu.sync_copy(ones_hbm, ones_v)
    plsc.subcore_barrier()

    def body(j, carry):
        pltpu.sync_copy(ones_v, acc.at[dst_v.at[j]], add=True)
        return carry

    lax.fori_loop(0, C, body, 0)
    plsc.subcore_barrier()
    pltpu.sync_copy(acc.at[pl.ds(r0, RPT)], out_hbm.at[cid, pl.ds(r0, RPT)])


@functools.partial(
    pl.kernel,
    out_type=_acc_ty,
    mesh=_mesh,
    scratch_types=[
        pltpu.VMEM((CA, CHUNK), jnp.int32),
        pltpu.VMEM((C, CHUNK), jnp.int32),
        pltpu.VMEM((NBUF, CHUNK, DH), jnp.float32),
        pltpu.VMEM_SHARED((N_ACC, DH), jnp.float32),
        pltpu.SemaphoreType.DMA,
        pltpu.SemaphoreType.DMA,
        pltpu.SemaphoreType.DMA,
        pltpu.SemaphoreType.DMA,
        pltpu.SemaphoreType.DMA,
        pltpu.SemaphoreType.DMA,
        pltpu.SemaphoreType.DMA,
        pltpu.SemaphoreType.DMA,
    ],
    compiler_params=_sc_params,
)
def _sc_aggregate(g_hbm, src_hbm, dst_hbm, zeros_hbm, out_hbm,
                  src_v, dst_v, rows, acc,
                  gs0, gs1, gs2, gs3, ss0, ss1, ss2, ss3):
    cid = lax.axis_index("c")
    sid = lax.axis_index("s")
    wid = sid * NC + cid
    r0 = sid * RPT
    gsems = (gs0, gs1, gs2, gs3)
    ssems = (ss0, ss1, ss2, ss3)
    pltpu.sync_copy(zeros_hbm.at[pl.ds(r0, RPT)], acc.at[pl.ds(r0, RPT)])
    pltpu.sync_copy(src_hbm.at[wid], src_v)
    pltpu.sync_copy(dst_hbm.at[wid], dst_v)
    plsc.subcore_barrier()

    for b in range(NBUF):
        pltpu.async_copy(g_hbm.at[src_v.at[b]], rows.at[b], gsems[b])

    def body(i, carry):
        j0 = i * NBUF
        for b in range(NBUF):
            j = j0 + b
            pltpu.make_async_copy(
                g_hbm.at[src_v.at[j]], rows.at[b], gsems[b]).wait()
            pltpu.async_copy(rows.at[b], acc.at[dst_v.at[j]], ssems[b],
                             add=True)
        for b in range(NBUF):
            j = j0 + b
            pltpu.make_async_copy(
                rows.at[b], acc.at[dst_v.at[j]], ssems[b]).wait()
            pltpu.async_copy(g_hbm.at[src_v.at[j + NBUF]], rows.at[b],
                             gsems[b])
        return carry

    lax.fori_loop(0, C // NBUF, body, 0)
    for b in range(NBUF):
        pltpu.make_async_copy(
            g_hbm.at[src_v.at[C + b]], rows.at[b], gsems[b]).wait()
    plsc.subcore_barrier()
    pltpu.sync_copy(acc.at[pl.ds(r0, RPT)], out_hbm.at[cid, pl.ds(r0, RPT)])


def _mm1_body(x_ref, w_ref, o_ref):
    o_ref[...] = jnp.dot(x_ref[...], w_ref[...],
                         preferred_element_type=jnp.float32)


def _scale_body(degp_ref, h_ref, dinv_ref, g_ref):
    deg = degp_ref[0, :N, :] + degp_ref[1, :N, :] + 1.0
    dinv = lax.rsqrt(deg)
    dinv_ref[...] = dinv
    g_ref[...] = h_ref[...] * dinv


def _mid_body(aggp_ref, g1_ref, dinv_ref, b1_ref, w2_ref, g2_ref):
    s = aggp_ref[0, :N, :] + aggp_ref[1, :N, :] + g1_ref[...]
    a1 = dinv_ref[...] * s + b1_ref[...]
    h = jnp.maximum(a1, 0.0)
    h2 = jnp.dot(h, w2_ref[...], preferred_element_type=jnp.float32)
    g2_ref[...] = h2 * dinv_ref[...]


def _out_body(aggp_ref, g2_ref, dinv_ref, b2_ref, o_ref):
    s = aggp_ref[0, :N, :] + aggp_ref[1, :N, :] + g2_ref[...]
    a = dinv_ref[...] * s + b2_ref[...]
    m = jnp.max(a, axis=1, keepdims=True)
    z = a - m
    o_ref[...] = z - jnp.log(jnp.sum(jnp.exp(z), axis=1, keepdims=True))


_f32 = jnp.float32


def kernel(x, edge_index, W1, b1, W2, b2):
    src = edge_index[0].astype(jnp.int32)
    dst = edge_index[1].astype(jnp.int32)
    pad = E_PAD - E
    src3 = jnp.concatenate([src, jnp.zeros((pad,), jnp.int32)])
    src3 = src3.reshape(NW, C, CHUNK)
    src3 = jnp.concatenate(
        [src3, jnp.zeros((NW, NBUF, CHUNK), jnp.int32)], axis=1)
    dst3 = jnp.concatenate([dst, jnp.full((pad,), N, jnp.int32)])
    dst3 = dst3.reshape(NW, C, CHUNK)
    zeros_acc = jnp.zeros((N_ACC, DH), _f32)
    ones_blk = jnp.ones((CHUNK, DH), _f32)
    b1r = b1.reshape(1, DH)
    b2r = b2.reshape(1, DH)

    degp = _sc_degree(dst3, ones_blk, zeros_acc)

    h1 = pl.pallas_call(
        _mm1_body,
        out_shape=jax.ShapeDtypeStruct((N, DH), _f32),
    )(x, W1)

    dinv, g1 = pl.pallas_call(
        _scale_body,
        out_shape=(jax.ShapeDtypeStruct((N, DH), _f32),
                   jax.ShapeDtypeStruct((N, DH), _f32)),
    )(degp, h1)

    aggp1 = _sc_aggregate(g1, src3, dst3, zeros_acc)

    g2 = pl.pallas_call(
        _mid_body,
        out_shape=jax.ShapeDtypeStruct((N, DH), _f32),
    )(aggp1, g1, dinv, b1r, W2)

    aggp2 = _sc_aggregate(g2, src3, dst3, zeros_acc)

    out = pl.pallas_call(
        _out_body,
        out_shape=jax.ShapeDtypeStruct((N, DH), _f32),
    )(aggp2, g2, dinv, b2r)

    return out


# speedup vs baseline: 2.3513x; 2.3513x over previous
import functools

import jax
import jax.numpy as jnp
from jax import lax
from jax.experimental import pallas as pl
from jax.experimental.pallas import tpu as pltpu
from jax.experimental.pallas import tpu_sc as plsc

N = 10000
E = 320000
D_IN = 128
DH = 16
NC = 2
NS = 16
NW = NC * NS
CHUNK = 128
C = 79
EPW = C * CHUNK
E_PAD = NW * EPW
N_ACC = 10112
RPT = N_ACC // NS

_mesh = plsc.VectorSubcoreMesh(core_axis_name="c", subcore_axis_name="s")
_acc_ty = jax.ShapeDtypeStruct((NC, N_ACC, DH), jnp.float32)
_sc_params = pltpu.CompilerParams(use_tc_tiling_on_sc=False)


@functools.partial(
    pl.kernel,
    out_type=_acc_ty,
    mesh=_mesh,
    scratch_types=[
        pltpu.VMEM((C, CHUNK), jnp.int32),
        pltpu.VMEM((CHUNK, DH), jnp.float32),
        pltpu.VMEM_SHARED((N_ACC, DH), jnp.float32),
    ],
    compiler_params=_sc_params,
)
def _sc_degree(dst_hbm, ones_hbm, zeros_hbm, out_hbm, dst_v, ones_v, acc):
    cid = lax.axis_index("c")
    sid = lax.axis_index("s")
    wid = sid * NC + cid
    r0 = sid * RPT
    pltpu.sync_copy(zeros_hbm.at[pl.ds(r0, RPT)], acc.at[pl.ds(r0, RPT)])
    pltpu.sync_copy(dst_hbm.at[wid], dst_v)
    pltpu.sync_copy(ones_hbm, ones_v)
    plsc.subcore_barrier()

    def body(j, carry):
        pltpu.sync_copy(ones_v, acc.at[dst_v.at[j]], add=True)
        return carry

    lax.fori_loop(0, C, body, 0)
    plsc.subcore_barrier()
    pltpu.sync_copy(acc.at[pl.ds(r0, RPT)], out_hbm.at[cid, pl.ds(r0, RPT)])


@functools.partial(
    pl.kernel,
    out_type=_acc_ty,
    mesh=_mesh,
    scratch_types=[
        pltpu.VMEM((C, CHUNK), jnp.int32),
        pltpu.VMEM((C, CHUNK), jnp.int32),
        pltpu.VMEM((CHUNK, DH), jnp.float32),
        pltpu.VMEM_SHARED((N_ACC, DH), jnp.float32),
        pltpu.VMEM_SHARED((N_ACC, DH), jnp.float32),
    ],
    compiler_params=_sc_params,
)
def _sc_aggregate(g_hbm, src_hbm, dst_hbm, zeros_hbm, out_hbm,
                  src_v, dst_v, rows_v, g_sp, acc):
    cid = lax.axis_index("c")
    sid = lax.axis_index("s")
    wid = sid * NC + cid
    r0 = sid * RPT
    pltpu.sync_copy(zeros_hbm.at[pl.ds(r0, RPT)], acc.at[pl.ds(r0, RPT)])
    pltpu.sync_copy(g_hbm.at[pl.ds(r0, RPT)], g_sp.at[pl.ds(r0, RPT)])
    pltpu.sync_copy(src_hbm.at[wid], src_v)
    pltpu.sync_copy(dst_hbm.at[wid], dst_v)
    plsc.subcore_barrier()

    def body(j, carry):
        pltpu.sync_copy(g_sp.at[src_v.at[j]], rows_v)
        pltpu.sync_copy(rows_v, acc.at[dst_v.at[j]], add=True)
        return carry

    lax.fori_loop(0, C, body, 0)
    plsc.subcore_barrier()
    pltpu.sync_copy(acc.at[pl.ds(r0, RPT)], out_hbm.at[cid, pl.ds(r0, RPT)])


def _mm1_body(x_ref, w_ref, o_ref):
    o_ref[...] = jnp.dot(x_ref[...], w_ref[...],
                         preferred_element_type=jnp.float32)


def _scale_body(degp_ref, h_ref, dinv_ref, g_ref):
    deg = degp_ref[0, :N, :] + degp_ref[1, :N, :] + 1.0
    dinv = lax.rsqrt(deg)
    dinv_ref[...] = dinv
    g_ref[:N, :] = h_ref[...] * dinv


def _mid_body(aggp_ref, g1_ref, dinv_ref, b1_ref, w2_ref, g2_ref):
    s = aggp_ref[0, :N, :] + aggp_ref[1, :N, :] + g1_ref[:N, :]
    a1 = dinv_ref[...] * s + b1_ref[...]
    h = jnp.maximum(a1, 0.0)
    h2 = jnp.dot(h, w2_ref[...], preferred_element_type=jnp.float32)
    g2_ref[:N, :] = h2 * dinv_ref[...]


def _out_body(aggp_ref, g2_ref, dinv_ref, b2_ref, o_ref):
    s = aggp_ref[0, :N, :] + aggp_ref[1, :N, :] + g2_ref[:N, :]
    a = dinv_ref[...] * s + b2_ref[...]
    m = jnp.max(a, axis=1, keepdims=True)
    z = a - m
    o_ref[...] = z - jnp.log(jnp.sum(jnp.exp(z), axis=1, keepdims=True))


_f32 = jnp.float32


def kernel(x, edge_index, W1, b1, W2, b2):
    src = edge_index[0].astype(jnp.int32)
    dst = edge_index[1].astype(jnp.int32)
    pad = E_PAD - E
    src3 = jnp.concatenate([src, jnp.zeros((pad,), jnp.int32)])
    src3 = src3.reshape(NW, C, CHUNK)
    dst3 = jnp.concatenate([dst, jnp.full((pad,), N, jnp.int32)])
    dst3 = dst3.reshape(NW, C, CHUNK)
    zeros_acc = jnp.zeros((N_ACC, DH), _f32)
    ones_blk = jnp.ones((CHUNK, DH), _f32)
    b1r = b1.reshape(1, DH)
    b2r = b2.reshape(1, DH)

    degp = _sc_degree(dst3, ones_blk, zeros_acc)

    h1 = pl.pallas_call(
        _mm1_body,
        out_shape=jax.ShapeDtypeStruct((N, DH), _f32),
    )(x, W1)

    dinv, g1 = pl.pallas_call(
        _scale_body,
        out_shape=(jax.ShapeDtypeStruct((N, DH), _f32),
                   jax.ShapeDtypeStruct((N_ACC, DH), _f32)),
    )(degp, h1)

    aggp1 = _sc_aggregate(g1, src3, dst3, zeros_acc)

    g2 = pl.pallas_call(
        _mid_body,
        out_shape=jax.ShapeDtypeStruct((N_ACC, DH), _f32),
    )(aggp1, g1, dinv, b1r, W2)

    aggp2 = _sc_aggregate(g2, src3, dst3, zeros_acc)

    out = pl.pallas_call(
        _out_body,
        out_shape=jax.ShapeDtypeStruct((N, DH), _f32),
    )(aggp2, g2, dinv, b2r)

    return out


# speedup vs baseline: 3.1933x; 1.3581x over previous
import functools

import jax
import jax.numpy as jnp
from jax import lax
from jax.experimental import pallas as pl
from jax.experimental.pallas import tpu as pltpu
from jax.experimental.pallas import tpu_sc as plsc

N = 10000
E = 320000
D_IN = 128
DH = 16
NC = 2
NS = 16
NW = NC * NS
CHUNK = 128
C = 79
EPW = C * CHUNK
E_PAD = NW * EPW
N_ACC = 10112
RPT = N_ACC // NS
PK = 8
NP = N // PK
NAP = N_ACC // PK

_mesh = plsc.VectorSubcoreMesh(core_axis_name="c", subcore_axis_name="s")
_acc_ty = jax.ShapeDtypeStruct((NC, N_ACC, DH), jnp.float32)
_sc_params = pltpu.CompilerParams(use_tc_tiling_on_sc=False)


@functools.partial(
    pl.kernel,
    out_type=_acc_ty,
    mesh=_mesh,
    scratch_types=[
        pltpu.VMEM((C, CHUNK), jnp.int32),
        pltpu.VMEM((CHUNK, DH), jnp.float32),
        pltpu.VMEM_SHARED((N_ACC, DH), jnp.float32),
    ],
    compiler_params=_sc_params,
)
def _sc_degree(dst_hbm, ones_hbm, zeros_hbm, out_hbm, dst_v, ones_v, acc):
    cid = lax.axis_index("c")
    sid = lax.axis_index("s")
    wid = sid * NC + cid
    r0 = sid * RPT
    pltpu.sync_copy(zeros_hbm.at[pl.ds(r0, RPT)], acc.at[pl.ds(r0, RPT)])
    pltpu.sync_copy(dst_hbm.at[wid], dst_v)
    pltpu.sync_copy(ones_hbm, ones_v)
    plsc.subcore_barrier()

    def body(j, carry):
        pltpu.sync_copy(ones_v, acc.at[dst_v.at[j]], add=True)
        return carry

    lax.fori_loop(0, C, body, 0)
    plsc.subcore_barrier()
    pltpu.sync_copy(acc.at[pl.ds(r0, RPT)], out_hbm.at[cid, pl.ds(r0, RPT)])


@functools.partial(
    pl.kernel,
    out_type=_acc_ty,
    mesh=_mesh,
    scratch_types=[
        pltpu.VMEM((C, CHUNK), jnp.int32),
        pltpu.VMEM((C, CHUNK), jnp.int32),
        pltpu.VMEM((CHUNK, DH), jnp.float32),
        pltpu.VMEM_SHARED((N_ACC, DH), jnp.float32),
        pltpu.VMEM_SHARED((N_ACC, DH), jnp.float32),
    ],
    compiler_params=_sc_params,
)
def _sc_aggregate(g_hbm, src_hbm, dst_hbm, zeros_hbm, out_hbm,
                  src_v, dst_v, rows_v, g_sp, acc):
    cid = lax.axis_index("c")
    sid = lax.axis_index("s")
    wid = sid * NC + cid
    r0 = sid * RPT
    pltpu.sync_copy(zeros_hbm.at[pl.ds(r0, RPT)], acc.at[pl.ds(r0, RPT)])
    pltpu.sync_copy(g_hbm.at[pl.ds(r0, RPT)], g_sp.at[pl.ds(r0, RPT)])
    pltpu.sync_copy(src_hbm.at[wid], src_v)
    pltpu.sync_copy(dst_hbm.at[wid], dst_v)
    plsc.subcore_barrier()

    def body(j, carry):
        pltpu.sync_copy(g_sp.at[src_v.at[j]], rows_v)
        pltpu.sync_copy(rows_v, acc.at[dst_v.at[j]], add=True)
        return carry

    lax.fori_loop(0, C, body, 0)
    plsc.subcore_barrier()
    pltpu.sync_copy(acc.at[pl.ds(r0, RPT)], out_hbm.at[cid, pl.ds(r0, RPT)])


def _scale_body(xr_ref, w1b_ref, degp_ref, dinv_ref, g_ref):
    h_p = jnp.dot(xr_ref[...], w1b_ref[...],
                  preferred_element_type=jnp.float32)
    deg = degp_ref[:NAP, :] + degp_ref[NAP:, :] + 1.0
    dinv = lax.rsqrt(deg)
    dinv_ref[...] = dinv
    g_ref[:NP, :] = h_p * dinv[:NP, :]


def _mid_body(aggp_ref, g1_ref, dinv_ref, b1_ref, w2b_ref, g2_ref):
    s = aggp_ref[:NAP, :] + aggp_ref[NAP:, :] + g1_ref[...]
    a1 = dinv_ref[...] * s + b1_ref[...]
    h = jnp.maximum(a1, 0.0)
    h2 = jnp.dot(h, w2b_ref[...], preferred_element_type=jnp.float32)
    g2_ref[:NP, :] = (h2 * dinv_ref[...])[:NP, :]


def _out_body(aggp_ref, g2_ref, dinv_ref, b2_ref, sb_ref, o_ref):
    s = (aggp_ref[:NP, :] + aggp_ref[NAP:NAP + NP, :] + g2_ref[:NP, :])
    a = dinv_ref[:NP, :] * s + b2_ref[...]
    lanes = lax.broadcasted_iota(jnp.int32, (NP, 128), 1)
    m = a
    for k in (1, 2, 4, 8):
        up = jnp.roll(m, -k, axis=1)
        dn = jnp.roll(m, k, axis=1)
        m = jnp.maximum(m, jnp.where((lanes & k) == 0, up, dn))
    z = a - m
    ez = jnp.exp(z)
    gs = jnp.dot(ez, sb_ref[...], preferred_element_type=jnp.float32)
    o_ref[...] = z - jnp.log(gs)


_f32 = jnp.float32


def kernel(x, edge_index, W1, b1, W2, b2):
    src = edge_index[0].astype(jnp.int32)
    dst = edge_index[1].astype(jnp.int32)
    pad = E_PAD - E
    src3 = jnp.concatenate([src, jnp.zeros((pad,), jnp.int32)])
    src3 = src3.reshape(NW, C, CHUNK)
    dst3 = jnp.concatenate([dst, jnp.full((pad,), N, jnp.int32)])
    dst3 = dst3.reshape(NW, C, CHUNK)
    zeros_acc = jnp.zeros((N_ACC, DH), _f32)
    ones_blk = jnp.ones((CHUNK, DH), _f32)
    W1B = jnp.kron(jnp.eye(PK, dtype=_f32), W1)
    W2B = jnp.kron(jnp.eye(PK, dtype=_f32), W2)
    SB = jnp.kron(jnp.eye(PK, dtype=_f32), jnp.ones((DH, DH), _f32))
    b1B = jnp.tile(b1, (PK,)).reshape(1, 128)
    b2B = jnp.tile(b2, (PK,)).reshape(1, 128)
    x_r = x.reshape(NP, PK * D_IN)

    degp = _sc_degree(dst3, ones_blk, zeros_acc)
    degp_p = degp.reshape(NC * NAP, 128)

    dinv_p, g1_p = pl.pallas_call(
        _scale_body,
        out_shape=(jax.ShapeDtypeStruct((NAP, 128), _f32),
                   jax.ShapeDtypeStruct((NAP, 128), _f32)),
    )(x_r, W1B, degp_p)

    aggp1 = _sc_aggregate(g1_p.reshape(N_ACC, DH), src3, dst3, zeros_acc)

    g2_p = pl.pallas_call(
        _mid_body,
        out_shape=jax.ShapeDtypeStruct((NAP, 128), _f32),
    )(aggp1.reshape(NC * NAP, 128), g1_p, dinv_p, b1B, W2B)

    aggp2 = _sc_aggregate(g2_p.reshape(N_ACC, DH), src3, dst3, zeros_acc)

    out_p = pl.pallas_call(
        _out_body,
        out_shape=jax.ShapeDtypeStruct((NP, 128), _f32),
    )(aggp2.reshape(NC * NAP, 128), g2_p, dinv_p, b2B, SB)

    return out_p.reshape(N, DH)
